# Initial kernel scaffold; baseline (speedup 1.0000x reference)
import jax
import jax.numpy as jnp
from jax.experimental import pallas as pl


def kernel(data, imps, flat_idx, batch):
    raise NotImplementedError("write your pallas kernel here")


# speedup vs baseline: 111.1030x; 111.1030x over previous
import functools

import jax
import jax.numpy as jnp
from jax import lax
from jax.experimental import pallas as pl
from jax.experimental.pallas import tpu as pltpu
from jax.experimental.pallas import tpu_sc as plsc

NC, NS, L = 2, 16, 16
NW = NC * NS
SENTINEL = 2**31 - 1


@functools.lru_cache(maxsize=None)
def _make_call(N, D, B, M, nnz, iters):
    RPW = B // NW
    C = 128
    NCH = RPW // C
    K = D // L + 1
    NG = C // L

    mesh = plsc.VectorSubcoreMesh(core_axis_name="c", subcore_axis_name="s")

    @functools.partial(
        pl.kernel,
        out_type=jax.ShapeDtypeStruct((B, D), jnp.float32),
        mesh=mesh,
        compiler_params=pltpu.CompilerParams(
            needs_layout_passes=False, use_tc_tiling_on_sc=False),
        scratch_types=[
            pltpu.VMEM((NCH, C), jnp.int32),
            pltpu.VMEM((NCH, C), jnp.int32),
            pltpu.VMEM((NCH, C), jnp.int32),
            pltpu.VMEM((NCH, C), jnp.int32),
            pltpu.VMEM((NCH, C, L), jnp.int32),
            pltpu.VMEM((K, C), jnp.int32),
            pltpu.VMEM((K, C, L), jnp.int32),
            pltpu.VMEM((K, C, L), jnp.float32),
            pltpu.VMEM((C, D), jnp.float32),
            pltpu.SemaphoreType.DMA,
            pltpu.SemaphoreType.DMA,
            pltpu.SemaphoreType.DMA,
            pltpu.SemaphoreType.DMA,
        ],
    )
    def call(data_hbm, flat_hbm, imps_hbm, batch_hbm, out_hbm,
             batch_v, lo_v, hi_v, probe_v, gbuf, segidx, fbuf, vbuf, rows_v,
             sem_b, sem_s, sem_w, sem_r):
        wid = lax.axis_index("s") * NC + lax.axis_index("c")
        base = pl.multiple_of(wid * RPW, RPW)

        for q in range(NCH):
            pltpu.async_copy(
                batch_hbm.at[pl.ds(base + q * C, C)], batch_v.at[q], sem_b)
        zero16 = jnp.zeros((L,), jnp.int32)
        nnz16 = jnp.full((L,), nnz, jnp.int32)
        for q in range(NCH):
            for g in range(NG):
                lo_v[q, pl.ds(g * L, L)] = zero16
                hi_v[q, pl.ds(g * L, L)] = nnz16
        for q in range(NCH):
            pltpu.make_async_copy(
                batch_hbm.at[pl.ds(base + q * C, C)], batch_v.at[q],
                sem_b).wait()

        iota16 = lax.iota(jnp.int32, L)

        def search_round(it, carry):
            for q in range(NCH):
                for g in range(NG):
                    lo = lo_v[q, pl.ds(g * L, L)]
                    hi = hi_v[q, pl.ds(g * L, L)]
                    mid = (lo + hi) >> 1
                    probe_v[q, pl.ds(g * L, L)] = mid >> 4
            for q in range(NCH):
                pltpu.async_copy(flat_hbm.at[probe_v.at[q]], gbuf.at[q], sem_s)
            for q in range(NCH):
                pltpu.make_async_copy(
                    flat_hbm.at[probe_v.at[q]], gbuf.at[q], sem_s).wait()
            for q in range(NCH):
                qv = jnp.full((L,), q, jnp.int32)
                for g in range(NG):
                    lo = lo_v[q, pl.ds(g * L, L)]
                    hi = hi_v[q, pl.ds(g * L, L)]
                    mid = (lo + hi) >> 1
                    rows = iota16 + (g * L)
                    val = plsc.load_gather(gbuf, [qv, rows, mid & 15])
                    tgt = batch_v[q, pl.ds(g * L, L)] << 7
                    pred = val < tgt
                    lo_v[q, pl.ds(g * L, L)] = jnp.where(pred, mid + 1, lo)
                    hi_v[q, pl.ds(g * L, L)] = jnp.where(pred, hi, mid)
            return carry
        lax.fori_loop(0, iters, search_round, 0)

        for q in range(NCH):
            for g in range(NG):
                sg = lo_v[q, pl.ds(g * L, L)] >> 4
                for k in range(K):
                    segidx[k, pl.ds(g * L, L)] = sg + k
            handles = [pltpu.async_copy(
                data_hbm.at[batch_v.at[q]], rows_v, sem_r)]
            for k in range(K):
                handles.append(pltpu.async_copy(
                    flat_hbm.at[segidx.at[k]], fbuf.at[k], sem_w))
                handles.append(pltpu.async_copy(
                    imps_hbm.at[segidx.at[k]], vbuf.at[k], sem_w))
            for h in handles:
                h.wait()

            for g in range(NG):
                rows16 = iota16 + (g * L)
                s_vec = lo_v[q, pl.ds(g * L, L)]
                sg = s_vec >> 4
                tgt = batch_v[q, pl.ds(g * L, L)] << 7

                def j_fn(j, carry, rows16=rows16, s_vec=s_vec, sg=sg,
                         tgt=tgt):
                    p = s_vec + j
                    gk = (p >> 4) - sg
                    ln = p & 15
                    f = plsc.load_gather(fbuf, [gk, rows16, ln])
                    v = plsc.load_gather(vbuf, [gk, rows16, ln])
                    local = f - tgt
                    plsc.store_scatter(rows_v, [rows16, local], v,
                                       mask=local < D)
                    return carry
                lax.fori_loop(0, D, j_fn, 0)

            pltpu.sync_copy(rows_v, out_hbm.at[pl.ds(base + q * C, C)])

    return call


def kernel(data, imps, flat_idx, batch):
    N, D = data.shape
    B = batch.shape[0]
    nnz = flat_idx.shape[0]
    M = (nnz + L - 1) // L + D // L + 1
    pad = M * L - nnz
    flat2d = jnp.concatenate(
        [flat_idx, jnp.full((pad,), SENTINEL, jnp.int32)]).reshape(M, L)
    imps2d = jnp.concatenate(
        [imps, jnp.zeros((pad,), jnp.float32)]).reshape(M, L)
    iters = max(1, int(nnz).bit_length())
    call = _make_call(N, D, B, M, nnz, iters)
    return call(data, flat2d, imps2d, batch)


# speedup vs baseline: 136.1908x; 1.2258x over previous
import functools

import jax
import jax.numpy as jnp
from jax import lax
from jax.experimental import pallas as pl
from jax.experimental.pallas import tpu as pltpu
from jax.experimental.pallas import tpu_sc as plsc

NC, NS, L = 2, 16, 16
NW = NC * NS
SENTINEL = 2**31 - 1
SLACK = 16


@functools.lru_cache(maxsize=None)
def _make_call(N, D, B, M, nnz, iters):
    RPW = B // NW
    C = 128
    NCH = RPW // C
    W = D + SLACK
    K = W // L + 1
    NG = C // L

    mesh = plsc.VectorSubcoreMesh(core_axis_name="c", subcore_axis_name="s")

    @functools.partial(
        pl.kernel,
        out_type=jax.ShapeDtypeStruct((B, D), jnp.float32),
        mesh=mesh,
        compiler_params=pltpu.CompilerParams(
            needs_layout_passes=False, use_tc_tiling_on_sc=False,
            disable_bounds_checks=True),
        scratch_types=[
            pltpu.VMEM((NCH, C), jnp.int32),
            pltpu.VMEM((NCH, C), jnp.int32),
            pltpu.VMEM((NCH, C), jnp.int32),
            pltpu.VMEM((NCH * 3, C), jnp.int32),
            pltpu.VMEM((NCH * 3, C, L), jnp.int32),
            pltpu.VMEM((K, C), jnp.int32),
            pltpu.VMEM((K, C, L), jnp.int32),
            pltpu.VMEM((K, C, L), jnp.float32),
            pltpu.VMEM((C, D), jnp.float32),
            pltpu.SemaphoreType.DMA,
            pltpu.SemaphoreType.DMA,
            pltpu.SemaphoreType.DMA,
            pltpu.SemaphoreType.DMA,
        ],
    )
    def call(data_hbm, flat_hbm, imps_hbm, batch_hbm, out_hbm,
             batch_v, lo_v, hi_v, probe_v, gbuf, segidx, fbuf, vbuf, rows_v,
             sem_b, sem_s, sem_w, sem_r):
        wid = lax.axis_index("s") * NC + lax.axis_index("c")
        base = pl.multiple_of(wid * RPW, RPW)

        for q in range(NCH):
            pltpu.async_copy(
                batch_hbm.at[pl.ds(base + q * C, C)], batch_v.at[q], sem_b)
        zero16 = jnp.zeros((L,), jnp.int32)
        nnz16 = jnp.full((L,), nnz, jnp.int32)
        for q in range(NCH):
            for g in range(NG):
                lo_v[q, pl.ds(g * L, L)] = zero16
                hi_v[q, pl.ds(g * L, L)] = nnz16
        for q in range(NCH):
            pltpu.make_async_copy(
                batch_hbm.at[pl.ds(base + q * C, C)], batch_v.at[q],
                sem_b).wait()

        iota16 = lax.iota(jnp.int32, L)

        def quartiles(lo, hi):
            qlen = (hi - lo) >> 2
            m1 = lo + qlen
            m2 = m1 + qlen
            m3 = m2 + qlen
            return m1, m2, m3

        def search_round(it, carry):
            for q in range(NCH):
                for g in range(NG):
                    sl = pl.ds(g * L, L)
                    m1, m2, m3 = quartiles(lo_v[q, sl], hi_v[q, sl])
                    probe_v[q * 3 + 0, sl] = m1 >> 4
                    probe_v[q * 3 + 1, sl] = m2 >> 4
                    probe_v[q * 3 + 2, sl] = m3 >> 4
            for j in range(NCH * 3):
                pltpu.async_copy(flat_hbm.at[probe_v.at[j]], gbuf.at[j], sem_s)
            for j in range(NCH * 3):
                pltpu.make_async_copy(
                    flat_hbm.at[probe_v.at[j]], gbuf.at[j], sem_s).wait()
            for q in range(NCH):
                for g in range(NG):
                    sl = pl.ds(g * L, L)
                    lo = lo_v[q, sl]
                    hi = hi_v[q, sl]
                    m1, m2, m3 = quartiles(lo, hi)
                    rows = iota16 + (g * L)
                    v1 = plsc.load_gather(
                        gbuf, [jnp.full((L,), q * 3 + 0, jnp.int32), rows,
                               m1 & 15])
                    v2 = plsc.load_gather(
                        gbuf, [jnp.full((L,), q * 3 + 1, jnp.int32), rows,
                               m2 & 15])
                    v3 = plsc.load_gather(
                        gbuf, [jnp.full((L,), q * 3 + 2, jnp.int32), rows,
                               m3 & 15])
                    tgt = batch_v[q, sl] << 7
                    p1 = v1 < tgt
                    p2 = v2 < tgt
                    p3 = v3 < tgt
                    lo_v[q, sl] = jnp.where(
                        p3, m3 + 1, jnp.where(p2, m2 + 1,
                                              jnp.where(p1, m1 + 1, lo)))
                    hi_v[q, sl] = jnp.where(
                        p1, jnp.where(p2, jnp.where(p3, hi, m3), m2), m1)
            return carry
        lax.fori_loop(0, iters, search_round, 0)

        for q in range(NCH):
            for g in range(NG):
                sg = lo_v[q, pl.ds(g * L, L)] >> 4
                for k in range(K):
                    segidx[k, pl.ds(g * L, L)] = sg + k
            handles = [pltpu.async_copy(
                data_hbm.at[batch_v.at[q]], rows_v, sem_r)]
            for k in range(K):
                handles.append(pltpu.async_copy(
                    flat_hbm.at[segidx.at[k]], fbuf.at[k], sem_w))
                handles.append(pltpu.async_copy(
                    imps_hbm.at[segidx.at[k]], vbuf.at[k], sem_w))
            for h in handles:
                h.wait()

            for g in range(NG):
                rows16 = iota16 + (g * L)
                s_vec = lo_v[q, pl.ds(g * L, L)]
                sg = s_vec >> 4
                tgt = batch_v[q, pl.ds(g * L, L)] << 7

                @plsc.parallel_loop(0, W, 1, unroll=8)
                def j_body(j, rows16=rows16, s_vec=s_vec, sg=sg, tgt=tgt):
                    p = s_vec + j
                    gk = (p >> 4) - sg
                    ln = p & 15
                    f = plsc.load_gather(fbuf, [gk, rows16, ln])
                    v = plsc.load_gather(vbuf, [gk, rows16, ln])
                    local = f - tgt
                    in_row = plsc.bitcast(local, jnp.uint32) < jnp.uint32(D)
                    plsc.store_scatter(rows_v, [rows16, local], v,
                                       mask=in_row)

            pltpu.sync_copy(rows_v, out_hbm.at[pl.ds(base + q * C, C)])

    return call


def kernel(data, imps, flat_idx, batch):
    N, D = data.shape
    B = batch.shape[0]
    nnz = flat_idx.shape[0]
    M = (nnz + L - 1) // L + (D + SLACK) // L + 1
    pad = M * L - nnz
    flat2d = jnp.concatenate(
        [flat_idx, jnp.full((pad,), SENTINEL, jnp.int32)]).reshape(M, L)
    imps2d = jnp.concatenate(
        [imps, jnp.zeros((pad,), jnp.float32)]).reshape(M, L)
    iters, blen = 0, nnz
    while blen > SLACK:
        blen = blen // 4 + 2
        iters += 1
    iters = max(1, iters)
    call = _make_call(N, D, B, M, nnz, iters)
    return call(data, flat2d, imps2d, batch)
